# Initial kernel scaffold; baseline (speedup 1.0000x reference)
import jax
import jax.numpy as jnp
from jax.experimental import pallas as pl


def kernel(xyz, points, t_embed, conv_w0, conv_b0, tw0, tb0, gamma0, beta0, conv_w1, conv_b1, tw1, tb1, gamma1, beta1, conv_w2, conv_b2, tw2, tb2, gamma2, beta2):
    raise NotImplementedError("write your pallas kernel here")


# speedup vs baseline: 12.1548x; 12.1548x over previous
import functools
import math

import jax
import jax.numpy as jnp
import numpy as np
from jax import lax
from jax.experimental import pallas as pl
from jax.experimental.pallas import tpu as pltpu
from jax.experimental.pallas import tpu_sc as plsc

B = 8
N = 4096
S = 1024
K = 32
R2 = 0.4 ** 2
C0 = 64
C1 = 64
C2 = 128
STILE = 128
RROWS = STILE * K
MTOT = B * S * K
NSTEP = S // STILE

_SQ = S // 4
_GCH = MTOT // 32 // 128


def _gelu(x):
    return x * 0.5 * (1.0 + lax.erf(x * np.float32(1.0 / math.sqrt(2.0))))


def _fps_body(x3_ref, far0_ref, nxt_ref):
    x = x3_ref[0]
    y = x3_ref[1]
    z = x3_ref[2]
    col = lax.broadcasted_iota(jnp.int32, (B, N), 1)
    scol = lax.broadcasted_iota(jnp.int32, (3, B, S), 2)

    def body(i, carry):
        dist, far = carry
        m = col == far
        cx = jnp.sum(jnp.where(m, x, 0.0), axis=1, keepdims=True)
        cy = jnp.sum(jnp.where(m, y, 0.0), axis=1, keepdims=True)
        cz = jnp.sum(jnp.where(m, z, 0.0), axis=1, keepdims=True)
        c3 = jnp.concatenate([cx[None], cy[None], cz[None]], axis=0)
        nxt_ref[...] = jnp.where(scol == i, jnp.broadcast_to(c3, (3, B, S)),
                                 nxt_ref[...])
        d = (x - cx) ** 2 + (y - cy) ** 2 + (z - cz) ** 2
        dist = jnp.minimum(dist, d)
        mx = jnp.max(dist, axis=1, keepdims=True)
        far = jnp.min(jnp.where(dist == mx, col, N), axis=1, keepdims=True)
        return dist, far

    dist0 = jnp.full((B, N), 1e10, dtype=jnp.float32)
    far0 = far0_ref[...]
    lax.fori_loop(0, S, body, (dist0, far0))


def _fps(x3, far0):
    return pl.pallas_call(
        _fps_body,
        out_shape=jax.ShapeDtypeStruct((3, B, S), jnp.float32),
    )(x3, far0)


def _z_body(xyz_ref, pts_ref, w_ref, z_ref):
    A = w_ref[:, :3]
    P = w_ref[:, 3:]
    xyzb = xyz_ref[0]
    ptsb = pts_ref[0]
    z1 = lax.dot_general(xyzb, A, (((1,), (1,)), ((), ())),
                         preferred_element_type=jnp.float32)
    z2 = lax.dot_general(ptsb, P, (((1,), (1,)), ((), ())),
                         preferred_element_type=jnp.float32)
    z_ref[0] = jnp.concatenate(
        [z1 + z2, jnp.zeros((N, 128 - C0), jnp.float32)], axis=1)


def _zcall(xyz, pts_t, cw0):
    return pl.pallas_call(
        _z_body,
        grid=(B,),
        in_specs=[
            pl.BlockSpec((1, N, 3), lambda b: (b, 0, 0)),
            pl.BlockSpec((1, N, 64), lambda b: (b, 0, 0)),
            pl.BlockSpec((C0, 67), lambda b: (0, 0)),
        ],
        out_specs=pl.BlockSpec((1, N, 128), lambda b: (b, 0, 0)),
        out_shape=jax.ShapeDtypeStruct((B, N, 128), jnp.float32),
    )(xyz, pts_t, cw0)


def _prep_body(te_ref, tw0_ref, tb0_ref, cw0_ref, cb0_ref,
               tw1_ref, tb1_ref, cw1_ref, cb1_ref,
               tw2_ref, tb2_ref, cw2_ref, cb2_ref,
               u0_ref, u1_ref, u2_ref):
    ge = _gelu(te_ref[...])

    def mk(tw, tb, cw, cb):
        t = lax.dot_general(ge, tw[...], (((1,), (1,)), ((), ())),
                            preferred_element_type=jnp.float32) + tb[...]
        return lax.dot_general(t, cw[...], (((1,), (1,)), ((), ())),
                               preferred_element_type=jnp.float32) + cb[...]

    u0_ref[...] = mk(tw0_ref, tb0_ref, cw0_ref, cb0_ref)
    u1_ref[...] = mk(tw1_ref, tb1_ref, cw1_ref, cb1_ref)
    u2_ref[...] = mk(tw2_ref, tb2_ref, cw2_ref, cb2_ref)


def _prep(te, tw0, tb0, cw0, cb0, tw1, tb1, cw1, cb1, tw2, tb2, cw2, cb2):
    return pl.pallas_call(
        _prep_body,
        out_shape=[
            jax.ShapeDtypeStruct((B, C0), jnp.float32),
            jax.ShapeDtypeStruct((B, C1), jnp.float32),
            jax.ShapeDtypeStruct((B, C2), jnp.float32),
        ],
    )(te, tw0, tb0.reshape(1, -1), cw0, cb0.reshape(1, -1),
      tw1, tb1.reshape(1, -1), cw1, cb1.reshape(1, -1),
      tw2, tb2.reshape(1, -1), cw2, cb2.reshape(1, -1))


def _bq_body(xyz_hbm, nxt_hbm, out_hbm, xv, yv, zv, cxs, cys, czs, idxv):
    c = lax.axis_index("c")
    s = lax.axis_index("s")
    wid = s * 2 + c
    b = wid // 4
    q = wid % 4
    pltpu.sync_copy(xyz_hbm.at[pl.dslice((b * 3 + 0) * N, N)], xv)
    pltpu.sync_copy(xyz_hbm.at[pl.dslice((b * 3 + 1) * N, N)], yv)
    pltpu.sync_copy(xyz_hbm.at[pl.dslice((b * 3 + 2) * N, N)], zv)
    pltpu.sync_copy(nxt_hbm.at[pl.dslice(0 * B * S + b * S + q * _SQ, _SQ)], cxs)
    pltpu.sync_copy(nxt_hbm.at[pl.dslice(1 * B * S + b * S + q * _SQ, _SQ)], cys)
    pltpu.sync_copy(nxt_hbm.at[pl.dslice(2 * B * S + b * S + q * _SQ, _SQ)], czs)
    lane = lax.iota(jnp.int32, 16)
    gbase = b * N

    def do_group(jg, _):
        cxv = cxs[pl.dslice(jg * 16, 16)]
        cyv = cys[pl.dslice(jg * 16, 16)]
        czv = czs[pl.dslice(jg * 16, 16)]
        for l in range(16):
            _one_cent(jg * 16 + l, cxv[l], cyv[l], czv[l])
        return 0

    def _one_cent(j, cx, cy, cz):
        def chunk(g, i, cnt):
            n0 = g * 16 + i
            xs = xv[pl.dslice(n0 * 16, 16)]
            ys = yv[pl.dslice(n0 * 16, 16)]
            zs = zv[pl.dslice(n0 * 16, 16)]
            dx = xs - cx
            dy = ys - cy
            dz = zs - cz
            d = dx * dx + dy * dy + dz * dz
            m = d <= R2
            pos = plsc.cumsum(m.astype(jnp.int32))
            wm = jnp.logical_and(m, (pos + cnt) <= K)
            dest = (pos + cnt - 1) + j * K
            gdx = lane + (n0 * 16 + gbase)
            plsc.store_scatter(idxv, [dest], gdx, mask=wm)
            cnt = cnt + jnp.sum(m.astype(jnp.int32))
            return cnt

        def sup(g, cnt):
            def work(c):
                return lax.fori_loop(0, 16, lambda i, c2: chunk(g, i, c2), c)
            return lax.cond(cnt < K, work, lambda c: c, cnt)

        cnt = lax.fori_loop(0, N // 256, sup, 0)
        cntc = jnp.minimum(cnt, K)
        row0 = idxv[pl.dslice(j * K, 16)]
        first = jnp.sum(jnp.where(lane == 0, row0, 0))
        for h in range(K // 16):
            cur = idxv[pl.dslice(j * K + h * 16, 16)]
            ln = lane + (h * 16)
            idxv[pl.dslice(j * K + h * 16, 16)] = jnp.where(ln < cntc, cur, first)

    lax.fori_loop(0, _SQ // 16, do_group, 0)
    pltpu.sync_copy(idxv, out_hbm.at[pl.dslice((b * S + q * _SQ) * K, _SQ * K)])


def _bq(xyz_t, nxt):
    mesh = plsc.VectorSubcoreMesh(core_axis_name="c", subcore_axis_name="s")
    fn = functools.partial(
        pl.kernel,
        mesh=mesh,
        out_type=jax.ShapeDtypeStruct((MTOT,), jnp.int32),
        compiler_params=pltpu.CompilerParams(needs_layout_passes=False),
        scratch_types=[
            pltpu.VMEM((N,), jnp.float32),
            pltpu.VMEM((N,), jnp.float32),
            pltpu.VMEM((N,), jnp.float32),
            pltpu.VMEM((_SQ,), jnp.float32),
            pltpu.VMEM((_SQ,), jnp.float32),
            pltpu.VMEM((_SQ,), jnp.float32),
            pltpu.VMEM((_SQ * K,), jnp.int32),
        ],
    )(_bq_body)
    return fn(xyz_t, nxt)


def _gather_body(z_hbm, gidx_hbm, out_hbm, idxv, buf, sem0, sem1):
    c = lax.axis_index("c")
    s = lax.axis_index("s")
    wid = s * 2 + c
    cbase = wid * _GCH
    pltpu.sync_copy(gidx_hbm.at[pl.dslice(cbase * 128, _GCH * 128)], idxv)

    def body(i, _):
        j0 = 2 * i
        j1 = 2 * i + 1
        cp0 = pltpu.async_copy(
            z_hbm.at[idxv.at[pl.dslice(j0 * 128, 128)]], buf.at[0], sem0)
        cp1 = pltpu.async_copy(
            z_hbm.at[idxv.at[pl.dslice(j1 * 128, 128)]], buf.at[1], sem1)
        cp0.wait()
        pltpu.sync_copy(buf.at[0],
                        out_hbm.at[pl.dslice((cbase + j0) * 128, 128)])
        cp1.wait()
        pltpu.sync_copy(buf.at[1],
                        out_hbm.at[pl.dslice((cbase + j1) * 128, 128)])
        return 0

    lax.fori_loop(0, _GCH // 2, body, 0)


def _gather(zflat, gidx2):
    mesh = plsc.VectorSubcoreMesh(core_axis_name="c", subcore_axis_name="s")
    fn = functools.partial(
        pl.kernel,
        mesh=mesh,
        out_type=jax.ShapeDtypeStruct((MTOT, 128), jnp.float32),
        compiler_params=pltpu.CompilerParams(needs_layout_passes=False),
        scratch_types=[
            pltpu.VMEM((_GCH * 128,), jnp.int32),
            pltpu.VMEM((2, 128, 128), jnp.float32),
            pltpu.SemaphoreType.DMA,
            pltpu.SemaphoreType.DMA,
        ],
    )(_gather_body)
    return fn(zflat, gidx2)


def _y0_tile(g_ref, nx_ref, w_ref, u_ref):
    Gt = g_ref[...][:, :C0]
    nx = nx_ref[0]
    A = w_ref[:, :3]
    v = u_ref[0] - lax.dot_general(nx, A, (((1,), (1,)), ((), ())),
                                     preferred_element_type=jnp.float32)
    vr = jnp.broadcast_to(v[:, None, :], (STILE, K, C0)).reshape(RROWS, C0)
    return Gt + vr


def _first_step():
    return jnp.logical_and(pl.program_id(0) == 0, pl.program_id(1) == 0)


def _stats0_body(g_ref, nx_ref, w_ref, u_ref, sum_ref):
    y0 = _y0_tile(g_ref, nx_ref, w_ref, u_ref)

    @pl.when(_first_step())
    def _():
        sum_ref[...] = jnp.zeros_like(sum_ref)

    s1 = jnp.sum(y0, axis=0, keepdims=True)
    s2 = jnp.sum(y0 * y0, axis=0, keepdims=True)
    sum_ref[...] += jnp.concatenate([s1, s2], axis=0)


def _stats0(g0, nx3, cw0, u0):
    return pl.pallas_call(
        _stats0_body,
        grid=(B, NSTEP),
        in_specs=[
            pl.BlockSpec((RROWS, 128), lambda b, j: (b * NSTEP + j, 0)),
            pl.BlockSpec((1, STILE, 3), lambda b, j: (b, j, 0)),
            pl.BlockSpec((C0, 67), lambda b, j: (0, 0)),
            pl.BlockSpec((1, 1, C0), lambda b, j: (b, 0, 0)),
        ],
        out_specs=pl.BlockSpec((2, C0), lambda b, j: (0, 0)),
        out_shape=jax.ShapeDtypeStruct((2, C0), jnp.float32),
    )(g0, nx3, cw0, u0.reshape(B, 1, C0))


def _l1_body(g_ref, nx_ref, w_ref, u_ref, sc_ref, sh_ref, w1_ref, u1_ref,
             y1_ref, sum_ref):
    y0 = _y0_tile(g_ref, nx_ref, w_ref, u_ref)
    x1 = _gelu(y0 * sc_ref[...] + sh_ref[...])
    y1 = lax.dot_general(x1, w1_ref[...], (((1,), (1,)), ((), ())),
                         preferred_element_type=jnp.float32) + u1_ref[0]
    y1_ref[...] = y1

    @pl.when(_first_step())
    def _():
        sum_ref[...] = jnp.zeros_like(sum_ref)

    s1 = jnp.sum(y1, axis=0, keepdims=True)
    s2 = jnp.sum(y1 * y1, axis=0, keepdims=True)
    sum_ref[...] += jnp.concatenate([s1, s2], axis=0)


def _l1(g0, nx3, cw0, u0, sc0, sh0, cw1, u1):
    return pl.pallas_call(
        _l1_body,
        grid=(B, NSTEP),
        in_specs=[
            pl.BlockSpec((RROWS, 128), lambda b, j: (b * NSTEP + j, 0)),
            pl.BlockSpec((1, STILE, 3), lambda b, j: (b, j, 0)),
            pl.BlockSpec((C0, 67), lambda b, j: (0, 0)),
            pl.BlockSpec((1, 1, C0), lambda b, j: (b, 0, 0)),
            pl.BlockSpec((1, C0), lambda b, j: (0, 0)),
            pl.BlockSpec((1, C0), lambda b, j: (0, 0)),
            pl.BlockSpec((C1, C0), lambda b, j: (0, 0)),
            pl.BlockSpec((1, 1, C1), lambda b, j: (b, 0, 0)),
        ],
        out_specs=[
            pl.BlockSpec((RROWS, C1), lambda b, j: (b * NSTEP + j, 0)),
            pl.BlockSpec((2, C1), lambda b, j: (0, 0)),
        ],
        out_shape=[
            jax.ShapeDtypeStruct((MTOT, C1), jnp.float32),
            jax.ShapeDtypeStruct((2, C1), jnp.float32),
        ],
    )(g0, nx3, cw0, u0.reshape(B, 1, C0), sc0, sh0, cw1, u1.reshape(B, 1, C1))


def _stats2_body(y1_ref, sc_ref, sh_ref, w2_ref, u2_ref, sum_ref):
    x2 = _gelu(y1_ref[...] * sc_ref[...] + sh_ref[...])
    y2 = lax.dot_general(x2, w2_ref[...], (((1,), (1,)), ((), ())),
                         preferred_element_type=jnp.float32) + u2_ref[0]

    @pl.when(_first_step())
    def _():
        sum_ref[...] = jnp.zeros_like(sum_ref)

    s1 = jnp.sum(y2, axis=0, keepdims=True)
    s2 = jnp.sum(y2 * y2, axis=0, keepdims=True)
    sum_ref[...] += jnp.concatenate([s1, s2], axis=0)


def _stats2(y1, sc1, sh1, cw2, u2):
    return pl.pallas_call(
        _stats2_body,
        grid=(B, NSTEP),
        in_specs=[
            pl.BlockSpec((RROWS, C1), lambda b, j: (b * NSTEP + j, 0)),
            pl.BlockSpec((1, C1), lambda b, j: (0, 0)),
            pl.BlockSpec((1, C1), lambda b, j: (0, 0)),
            pl.BlockSpec((C2, C1), lambda b, j: (0, 0)),
            pl.BlockSpec((1, 1, C2), lambda b, j: (b, 0, 0)),
        ],
        out_specs=pl.BlockSpec((2, C2), lambda b, j: (0, 0)),
        out_shape=jax.ShapeDtypeStruct((2, C2), jnp.float32),
    )(y1, sc1, sh1, cw2, u2.reshape(B, 1, C2))


def _final_body(y1_ref, sc_ref, sh_ref, w2_ref, u2_ref, sc2_ref, sh2_ref,
                out_ref):
    x2 = _gelu(y1_ref[...] * sc_ref[...] + sh_ref[...])
    y2 = lax.dot_general(x2, w2_ref[...], (((1,), (1,)), ((), ())),
                         preferred_element_type=jnp.float32) + u2_ref[0]
    z = _gelu(y2 * sc2_ref[...] + sh2_ref[...])
    out_ref[...] = jnp.max(z.reshape(STILE, K, C2), axis=1)


def _final(y1, sc1, sh1, cw2, u2, sc2, sh2):
    return pl.pallas_call(
        _final_body,
        grid=(B, NSTEP),
        in_specs=[
            pl.BlockSpec((RROWS, C1), lambda b, j: (b * NSTEP + j, 0)),
            pl.BlockSpec((1, C1), lambda b, j: (0, 0)),
            pl.BlockSpec((1, C1), lambda b, j: (0, 0)),
            pl.BlockSpec((C2, C1), lambda b, j: (0, 0)),
            pl.BlockSpec((1, 1, C2), lambda b, j: (b, 0, 0)),
            pl.BlockSpec((1, C2), lambda b, j: (0, 0)),
            pl.BlockSpec((1, C2), lambda b, j: (0, 0)),
        ],
        out_specs=pl.BlockSpec((STILE, C2), lambda b, j: (b * NSTEP + j, 0)),
        out_shape=jax.ShapeDtypeStruct((B * S, C2), jnp.float32),
    )(y1, sc1, sh1, cw2, u2.reshape(B, 1, C2), sc2, sh2)


def _scale_shift(sums, gamma, beta):
    mean = sums[0] / MTOT
    var = sums[1] / MTOT - mean * mean
    sc = gamma / jnp.sqrt(var + 1e-5)
    sh = beta - mean * sc
    return sc.reshape(1, -1), sh.reshape(1, -1)


def kernel(xyz, points, t_embed,
           conv_w0, conv_b0, tw0, tb0, gamma0, beta0,
           conv_w1, conv_b1, tw1, tb1, gamma1, beta1,
           conv_w2, conv_b2, tw2, tb2, gamma2, beta2):
    x3 = jnp.transpose(xyz, (2, 0, 1))
    xyz_t = jnp.transpose(xyz, (0, 2, 1))
    pts_t = jnp.transpose(points, (0, 2, 1))
    far0 = jax.random.randint(jax.random.key(42), (B,), 0, N).reshape(B, 1)

    nxt = _fps(x3, far0.astype(jnp.int32))
    nx3 = jnp.transpose(nxt, (1, 2, 0))
    z = _zcall(xyz, pts_t, conv_w0)
    u0, u1, u2 = _prep(t_embed, tw0, tb0, conv_w0, conv_b0,
                       tw1, tb1, conv_w1, conv_b1,
                       tw2, tb2, conv_w2, conv_b2)

    gidx = _bq(xyz_t.reshape(-1), nxt.reshape(-1))
    g0 = _gather(z.reshape(B * N, 128), gidx)

    sums0 = _stats0(g0, nx3, conv_w0, u0)
    sc0, sh0 = _scale_shift(sums0, gamma0, beta0)
    y1, sums1 = _l1(g0, nx3, conv_w0, u0, sc0, sh0, conv_w1, u1)
    sc1, sh1 = _scale_shift(sums1, gamma1, beta1)
    sums2 = _stats2(y1, sc1, sh1, conv_w2, u2)
    sc2, sh2 = _scale_shift(sums2, gamma2, beta2)
    out = _final(y1, sc1, sh1, conv_w2, u2, sc2, sh2)

    new_points = jnp.transpose(out.reshape(B, S, C2), (0, 2, 1))
    return nx3, new_points


# speedup vs baseline: 14.9037x; 1.2262x over previous
import functools
import math

import jax
import jax.numpy as jnp
import numpy as np
from jax import lax
from jax.experimental import pallas as pl
from jax.experimental.pallas import tpu as pltpu
from jax.experimental.pallas import tpu_sc as plsc

B = 8
N = 4096
S = 1024
K = 32
R2 = 0.4 ** 2
C0 = 64
C1 = 64
C2 = 128
STILE = 128
RROWS = STILE * K
MTOT = B * S * K
NSTEP = S // STILE

_SQ = S // 4
_GCH = MTOT // 32 // 128
_KP = K + 16


def _gelu(x):
    return x * 0.5 * (1.0 + lax.erf(x * np.float32(1.0 / math.sqrt(2.0))))


def _fps_body(x3_ref, far0_ref, nxt_ref):
    x = x3_ref[0]
    y = x3_ref[1]
    z = x3_ref[2]
    col = lax.broadcasted_iota(jnp.int32, (B, N), 1)
    scol = lax.broadcasted_iota(jnp.int32, (3, B, S), 2)

    def body(i, carry):
        dist, far = carry
        m = col == far
        cx = jnp.sum(jnp.where(m, x, 0.0), axis=1, keepdims=True)
        cy = jnp.sum(jnp.where(m, y, 0.0), axis=1, keepdims=True)
        cz = jnp.sum(jnp.where(m, z, 0.0), axis=1, keepdims=True)
        c3 = jnp.concatenate([cx[None], cy[None], cz[None]], axis=0)
        nxt_ref[...] = jnp.where(scol == i, jnp.broadcast_to(c3, (3, B, S)),
                                 nxt_ref[...])
        d = (x - cx) ** 2 + (y - cy) ** 2 + (z - cz) ** 2
        dist = jnp.minimum(dist, d)
        mx = jnp.max(dist, axis=1, keepdims=True)
        far = jnp.min(jnp.where(dist == mx, col, N), axis=1, keepdims=True)
        return dist, far

    dist0 = jnp.full((B, N), 1e10, dtype=jnp.float32)
    far0 = far0_ref[...]
    lax.fori_loop(0, S, body, (dist0, far0))


def _fps(x3, far0):
    return pl.pallas_call(
        _fps_body,
        out_shape=jax.ShapeDtypeStruct((3, B, S), jnp.float32),
    )(x3, far0)


def _z_body(xyz_ref, pts_ref, w_ref, z_ref):
    A = w_ref[:, :3]
    P = w_ref[:, 3:]
    xyzb = xyz_ref[0]
    ptsb = pts_ref[0]
    z1 = lax.dot_general(xyzb, A, (((1,), (1,)), ((), ())),
                         preferred_element_type=jnp.float32)
    z2 = lax.dot_general(ptsb, P, (((1,), (1,)), ((), ())),
                         preferred_element_type=jnp.float32)
    z_ref[0] = jnp.concatenate(
        [z1 + z2, jnp.zeros((N, 128 - C0), jnp.float32)], axis=1)


def _zcall(xyz, pts_t, cw0):
    return pl.pallas_call(
        _z_body,
        grid=(B,),
        in_specs=[
            pl.BlockSpec((1, N, 3), lambda b: (b, 0, 0)),
            pl.BlockSpec((1, N, 64), lambda b: (b, 0, 0)),
            pl.BlockSpec((C0, 67), lambda b: (0, 0)),
        ],
        out_specs=pl.BlockSpec((1, N, 128), lambda b: (b, 0, 0)),
        out_shape=jax.ShapeDtypeStruct((B, N, 128), jnp.float32),
    )(xyz, pts_t, cw0)


def _prep_body(te_ref, tw0_ref, tb0_ref, cw0_ref, cb0_ref,
               tw1_ref, tb1_ref, cw1_ref, cb1_ref,
               tw2_ref, tb2_ref, cw2_ref, cb2_ref,
               u0_ref, u1_ref, u2_ref):
    ge = _gelu(te_ref[...])

    def mk(tw, tb, cw, cb):
        t = lax.dot_general(ge, tw[...], (((1,), (1,)), ((), ())),
                            preferred_element_type=jnp.float32) + tb[...]
        return lax.dot_general(t, cw[...], (((1,), (1,)), ((), ())),
                               preferred_element_type=jnp.float32) + cb[...]

    u0_ref[...] = mk(tw0_ref, tb0_ref, cw0_ref, cb0_ref)
    u1_ref[...] = mk(tw1_ref, tb1_ref, cw1_ref, cb1_ref)
    u2_ref[...] = mk(tw2_ref, tb2_ref, cw2_ref, cb2_ref)


def _prep(te, tw0, tb0, cw0, cb0, tw1, tb1, cw1, cb1, tw2, tb2, cw2, cb2):
    return pl.pallas_call(
        _prep_body,
        out_shape=[
            jax.ShapeDtypeStruct((B, C0), jnp.float32),
            jax.ShapeDtypeStruct((B, C1), jnp.float32),
            jax.ShapeDtypeStruct((B, C2), jnp.float32),
        ],
    )(te, tw0, tb0.reshape(1, -1), cw0, cb0.reshape(1, -1),
      tw1, tb1.reshape(1, -1), cw1, cb1.reshape(1, -1),
      tw2, tb2.reshape(1, -1), cw2, cb2.reshape(1, -1))


def _bq_body(xyz_hbm, nxt_hbm, out_hbm, xv, yv, zv, cxs, cys, czs, idxv, obuf):
    c = lax.axis_index("c")
    s = lax.axis_index("s")
    wid = s * 2 + c
    b = wid // 4
    q = wid % 4
    pltpu.sync_copy(xyz_hbm.at[pl.dslice((b * 3 + 0) * N, N)], xv)
    pltpu.sync_copy(xyz_hbm.at[pl.dslice((b * 3 + 1) * N, N)], yv)
    pltpu.sync_copy(xyz_hbm.at[pl.dslice((b * 3 + 2) * N, N)], zv)
    pltpu.sync_copy(nxt_hbm.at[pl.dslice(0 * B * S + b * S + q * _SQ, _SQ)], cxs)
    pltpu.sync_copy(nxt_hbm.at[pl.dslice(1 * B * S + b * S + q * _SQ, _SQ)], cys)
    pltpu.sync_copy(nxt_hbm.at[pl.dslice(2 * B * S + b * S + q * _SQ, _SQ)], czs)
    lane = lax.iota(jnp.int32, 16)
    gbase = b * N

    def do_group(jg, _):
        cxv = cxs[pl.dslice(jg * 16, 16)]
        cyv = cys[pl.dslice(jg * 16, 16)]
        czv = czs[pl.dslice(jg * 16, 16)]
        for l in range(16):
            _one_cent(jg * 16 + l, cxv[l], cyv[l], czv[l])
        return 0

    def _one_cent(j, cx, cy, cz):
        def chunk(g, i, cnt):
            n0 = g * 16 + i
            xs = xv[pl.dslice(n0 * 16, 16)]
            ys = yv[pl.dslice(n0 * 16, 16)]
            zs = zv[pl.dslice(n0 * 16, 16)]
            dx = xs - cx
            dy = ys - cy
            dz = zs - cz
            d = dx * dx + dy * dy + dz * dz
            m = d <= R2
            gdx = lane + (n0 * 16 + gbase)
            off = jnp.minimum(cnt, K)
            plsc.store_compressed(idxv.at[pl.dslice(j * _KP + off, 16)],
                                  gdx, mask=m)
            return cnt + plsc.all_reduce_population_count(m)[0]

        def sup(g, cnt):
            def work(c):
                return lax.fori_loop(0, 16, lambda i, c2: chunk(g, i, c2), c)
            return lax.cond(cnt < K, work, lambda c: c, cnt)

        cnt = lax.fori_loop(0, N // 256, sup, 0)
        cntc = jnp.minimum(cnt, K)
        row0 = idxv[pl.dslice(j * _KP, 16)]
        first = jnp.sum(jnp.where(lane == 0, row0, 0))
        for h in range(K // 16):
            cur = idxv[pl.dslice(j * _KP + h * 16, 16)]
            ln = lane + (h * 16)
            obuf[pl.dslice(j * K + h * 16, 16)] = jnp.where(ln < cntc, cur, first)

    lax.fori_loop(0, _SQ // 16, do_group, 0)
    pltpu.sync_copy(obuf, out_hbm.at[pl.dslice((b * S + q * _SQ) * K, _SQ * K)])


def _bq(xyz_t, nxt):
    mesh = plsc.VectorSubcoreMesh(core_axis_name="c", subcore_axis_name="s")
    fn = functools.partial(
        pl.kernel,
        mesh=mesh,
        out_type=jax.ShapeDtypeStruct((MTOT,), jnp.int32),
        compiler_params=pltpu.CompilerParams(needs_layout_passes=False),
        scratch_types=[
            pltpu.VMEM((N,), jnp.float32),
            pltpu.VMEM((N,), jnp.float32),
            pltpu.VMEM((N,), jnp.float32),
            pltpu.VMEM((_SQ,), jnp.float32),
            pltpu.VMEM((_SQ,), jnp.float32),
            pltpu.VMEM((_SQ,), jnp.float32),
            pltpu.VMEM((_SQ * _KP,), jnp.int32),
            pltpu.VMEM((_SQ * K,), jnp.int32),
        ],
    )(_bq_body)
    return fn(xyz_t, nxt)


def _gather_body(z_hbm, gidx_hbm, out_hbm, idxv, buf,
                 gs0, gs1, gs2, gs3, os0, os1, os2, os3):
    c = lax.axis_index("c")
    s = lax.axis_index("s")
    wid = s * 2 + c
    cbase = wid * _GCH
    pltpu.sync_copy(gidx_hbm.at[pl.dslice(cbase * 128, _GCH * 128)], idxv)
    gsems = [gs0, gs1, gs2, gs3]
    osems = [os0, os1, os2, os3]

    def gsrc(j):
        return z_hbm.at[idxv.at[pl.dslice(j * 128, 128)]]

    def osrc(slot):
        return buf.at[slot]

    def odst(j):
        return out_hbm.at[pl.dslice((cbase + j) * 128, 128)]

    for slot in range(4):
        pltpu.async_copy(gsrc(slot), buf.at[slot], gsems[slot])

    def body(i, _):
        for slot in range(4):
            j = 4 * i + slot
            pltpu.make_async_copy(gsrc(j), buf.at[slot], gsems[slot]).wait()
            pltpu.async_copy(osrc(slot), odst(j), osems[slot])
            nj = j + 4

            @pl.when(nj < _GCH)
            def _():
                pltpu.make_async_copy(osrc(slot), odst(j), osems[slot]).wait()
                pltpu.async_copy(gsrc(nj), buf.at[slot], gsems[slot])
        return 0

    lax.fori_loop(0, _GCH // 4, body, 0)
    for slot in range(4):
        j = _GCH - 4 + slot
        pltpu.make_async_copy(osrc(slot), odst(j), osems[slot]).wait()


def _gather(zflat, gidx2):
    mesh = plsc.VectorSubcoreMesh(core_axis_name="c", subcore_axis_name="s")
    fn = functools.partial(
        pl.kernel,
        mesh=mesh,
        out_type=jax.ShapeDtypeStruct((MTOT, 128), jnp.float32),
        compiler_params=pltpu.CompilerParams(needs_layout_passes=False),
        scratch_types=[
            pltpu.VMEM((_GCH * 128,), jnp.int32),
            pltpu.VMEM((4, 128, 128), jnp.float32),
            pltpu.SemaphoreType.DMA,
            pltpu.SemaphoreType.DMA,
            pltpu.SemaphoreType.DMA,
            pltpu.SemaphoreType.DMA,
            pltpu.SemaphoreType.DMA,
            pltpu.SemaphoreType.DMA,
            pltpu.SemaphoreType.DMA,
            pltpu.SemaphoreType.DMA,
        ],
    )(_gather_body)
    return fn(zflat, gidx2)


def _y0_tile(g_ref, nx_ref, w_ref, u_ref):
    Gt = g_ref[...][:, :C0]
    nx = nx_ref[0]
    A = w_ref[:, :3]
    v = u_ref[0] - lax.dot_general(nx, A, (((1,), (1,)), ((), ())),
                                     preferred_element_type=jnp.float32)
    vr = jnp.broadcast_to(v[:, None, :], (STILE, K, C0)).reshape(RROWS, C0)
    return Gt + vr


def _first_step():
    return jnp.logical_and(pl.program_id(0) == 0, pl.program_id(1) == 0)


def _stats0_body(g_ref, nx_ref, w_ref, u_ref, sum_ref):
    y0 = _y0_tile(g_ref, nx_ref, w_ref, u_ref)

    @pl.when(_first_step())
    def _():
        sum_ref[...] = jnp.zeros_like(sum_ref)

    s1 = jnp.sum(y0, axis=0, keepdims=True)
    s2 = jnp.sum(y0 * y0, axis=0, keepdims=True)
    sum_ref[...] += jnp.concatenate([s1, s2], axis=0)


def _stats0(g0, nx3, cw0, u0):
    return pl.pallas_call(
        _stats0_body,
        grid=(B, NSTEP),
        in_specs=[
            pl.BlockSpec((RROWS, 128), lambda b, j: (b * NSTEP + j, 0)),
            pl.BlockSpec((1, STILE, 3), lambda b, j: (b, j, 0)),
            pl.BlockSpec((C0, 67), lambda b, j: (0, 0)),
            pl.BlockSpec((1, 1, C0), lambda b, j: (b, 0, 0)),
        ],
        out_specs=pl.BlockSpec((2, C0), lambda b, j: (0, 0)),
        out_shape=jax.ShapeDtypeStruct((2, C0), jnp.float32),
    )(g0, nx3, cw0, u0.reshape(B, 1, C0))


def _l1_body(g_ref, nx_ref, w_ref, u_ref, sc_ref, sh_ref, w1_ref, u1_ref,
             y1_ref, sum_ref):
    y0 = _y0_tile(g_ref, nx_ref, w_ref, u_ref)
    x1 = _gelu(y0 * sc_ref[...] + sh_ref[...])
    y1 = lax.dot_general(x1, w1_ref[...], (((1,), (1,)), ((), ())),
                         preferred_element_type=jnp.float32) + u1_ref[0]
    y1_ref[...] = y1

    @pl.when(_first_step())
    def _():
        sum_ref[...] = jnp.zeros_like(sum_ref)

    s1 = jnp.sum(y1, axis=0, keepdims=True)
    s2 = jnp.sum(y1 * y1, axis=0, keepdims=True)
    sum_ref[...] += jnp.concatenate([s1, s2], axis=0)


def _l1(g0, nx3, cw0, u0, sc0, sh0, cw1, u1):
    return pl.pallas_call(
        _l1_body,
        grid=(B, NSTEP),
        in_specs=[
            pl.BlockSpec((RROWS, 128), lambda b, j: (b * NSTEP + j, 0)),
            pl.BlockSpec((1, STILE, 3), lambda b, j: (b, j, 0)),
            pl.BlockSpec((C0, 67), lambda b, j: (0, 0)),
            pl.BlockSpec((1, 1, C0), lambda b, j: (b, 0, 0)),
            pl.BlockSpec((1, C0), lambda b, j: (0, 0)),
            pl.BlockSpec((1, C0), lambda b, j: (0, 0)),
            pl.BlockSpec((C1, C0), lambda b, j: (0, 0)),
            pl.BlockSpec((1, 1, C1), lambda b, j: (b, 0, 0)),
        ],
        out_specs=[
            pl.BlockSpec((RROWS, C1), lambda b, j: (b * NSTEP + j, 0)),
            pl.BlockSpec((2, C1), lambda b, j: (0, 0)),
        ],
        out_shape=[
            jax.ShapeDtypeStruct((MTOT, C1), jnp.float32),
            jax.ShapeDtypeStruct((2, C1), jnp.float32),
        ],
    )(g0, nx3, cw0, u0.reshape(B, 1, C0), sc0, sh0, cw1, u1.reshape(B, 1, C1))


def _stats2_body(y1_ref, sc_ref, sh_ref, w2_ref, u2_ref, sum_ref):
    x2 = _gelu(y1_ref[...] * sc_ref[...] + sh_ref[...])
    y2 = lax.dot_general(x2, w2_ref[...], (((1,), (1,)), ((), ())),
                         preferred_element_type=jnp.float32) + u2_ref[0]

    @pl.when(_first_step())
    def _():
        sum_ref[...] = jnp.zeros_like(sum_ref)

    s1 = jnp.sum(y2, axis=0, keepdims=True)
    s2 = jnp.sum(y2 * y2, axis=0, keepdims=True)
    sum_ref[...] += jnp.concatenate([s1, s2], axis=0)


def _stats2(y1, sc1, sh1, cw2, u2):
    return pl.pallas_call(
        _stats2_body,
        grid=(B, NSTEP),
        in_specs=[
            pl.BlockSpec((RROWS, C1), lambda b, j: (b * NSTEP + j, 0)),
            pl.BlockSpec((1, C1), lambda b, j: (0, 0)),
            pl.BlockSpec((1, C1), lambda b, j: (0, 0)),
            pl.BlockSpec((C2, C1), lambda b, j: (0, 0)),
            pl.BlockSpec((1, 1, C2), lambda b, j: (b, 0, 0)),
        ],
        out_specs=pl.BlockSpec((2, C2), lambda b, j: (0, 0)),
        out_shape=jax.ShapeDtypeStruct((2, C2), jnp.float32),
    )(y1, sc1, sh1, cw2, u2.reshape(B, 1, C2))


def _final_body(y1_ref, sc_ref, sh_ref, w2_ref, u2_ref, sc2_ref, sh2_ref,
                out_ref):
    x2 = _gelu(y1_ref[...] * sc_ref[...] + sh_ref[...])
    y2 = lax.dot_general(x2, w2_ref[...], (((1,), (1,)), ((), ())),
                         preferred_element_type=jnp.float32) + u2_ref[0]
    z = _gelu(y2 * sc2_ref[...] + sh2_ref[...])
    out_ref[...] = jnp.max(z.reshape(STILE, K, C2), axis=1)


def _final(y1, sc1, sh1, cw2, u2, sc2, sh2):
    return pl.pallas_call(
        _final_body,
        grid=(B, NSTEP),
        in_specs=[
            pl.BlockSpec((RROWS, C1), lambda b, j: (b * NSTEP + j, 0)),
            pl.BlockSpec((1, C1), lambda b, j: (0, 0)),
            pl.BlockSpec((1, C1), lambda b, j: (0, 0)),
            pl.BlockSpec((C2, C1), lambda b, j: (0, 0)),
            pl.BlockSpec((1, 1, C2), lambda b, j: (b, 0, 0)),
            pl.BlockSpec((1, C2), lambda b, j: (0, 0)),
            pl.BlockSpec((1, C2), lambda b, j: (0, 0)),
        ],
        out_specs=pl.BlockSpec((STILE, C2), lambda b, j: (b * NSTEP + j, 0)),
        out_shape=jax.ShapeDtypeStruct((B * S, C2), jnp.float32),
    )(y1, sc1, sh1, cw2, u2.reshape(B, 1, C2), sc2, sh2)


def _scale_shift(sums, gamma, beta):
    mean = sums[0] / MTOT
    var = sums[1] / MTOT - mean * mean
    sc = gamma / jnp.sqrt(var + 1e-5)
    sh = beta - mean * sc
    return sc.reshape(1, -1), sh.reshape(1, -1)


def kernel(xyz, points, t_embed,
           conv_w0, conv_b0, tw0, tb0, gamma0, beta0,
           conv_w1, conv_b1, tw1, tb1, gamma1, beta1,
           conv_w2, conv_b2, tw2, tb2, gamma2, beta2):
    x3 = jnp.transpose(xyz, (2, 0, 1))
    xyz_t = jnp.transpose(xyz, (0, 2, 1))
    pts_t = jnp.transpose(points, (0, 2, 1))
    far0 = jax.random.randint(jax.random.key(42), (B,), 0, N).reshape(B, 1)

    nxt = _fps(x3, far0.astype(jnp.int32))
    nx3 = jnp.transpose(nxt, (1, 2, 0))
    z = _zcall(xyz, pts_t, conv_w0)
    u0, u1, u2 = _prep(t_embed, tw0, tb0, conv_w0, conv_b0,
                       tw1, tb1, conv_w1, conv_b1,
                       tw2, tb2, conv_w2, conv_b2)

    gidx = _bq(xyz_t.reshape(-1), nxt.reshape(-1))
    g0 = _gather(z.reshape(B * N, 128), gidx)

    sums0 = _stats0(g0, nx3, conv_w0, u0)
    sc0, sh0 = _scale_shift(sums0, gamma0, beta0)
    y1, sums1 = _l1(g0, nx3, conv_w0, u0, sc0, sh0, conv_w1, u1)
    sc1, sh1 = _scale_shift(sums1, gamma1, beta1)
    sums2 = _stats2(y1, sc1, sh1, conv_w2, u2)
    sc2, sh2 = _scale_shift(sums2, gamma2, beta2)
    out = _final(y1, sc1, sh1, conv_w2, u2, sc2, sh2)

    new_points = jnp.transpose(out.reshape(B, S, C2), (0, 2, 1))
    return nx3, new_points
